# baseline (speedup 1.0000x reference)
import functools

import jax
import jax.numpy as jnp
from jax import lax
from jax.experimental import pallas as pl
from jax.experimental.pallas import tpu as pltpu
from jax.experimental.pallas import tpu_sc as plsc

N = 10000
E = 160000
G = 64
F0 = 28
H = 256
HH = 128
W1 = 32
NPAD = 10112
RPT = NPAD // 16
NTILES = 16
BLK = 128
EB = 80
EPT = E // NTILES
EB1 = 40
EPT1 = E // 32

_f32 = jnp.float32
_i32 = jnp.int32


def _sc_mesh():
    return plsc.VectorSubcoreMesh(core_axis_name="c", subcore_axis_name="s",
                                  num_cores=2, num_subcores=16)


@functools.cache
def _get_sc_agg():
    return functools.partial(
        pl.kernel,
        mesh=_sc_mesh(),
        out_type=(
            jax.ShapeDtypeStruct((NPAD, HH), _f32),
            jax.ShapeDtypeStruct((NPAD, HH), _f32),
        ),
        scratch_types=[
            pltpu.VMEM((EB, BLK), _i32),
            pltpu.VMEM((EB, BLK), _i32),
            pltpu.VMEM((BLK, HH), _f32),
            pltpu.VMEM_SHARED((NPAD, HH), _f32),
            pltpu.SemaphoreType.DMA,
        ],
    )(_sc_agg_body)


def _sc_agg(*args):
    return _get_sc_agg()(*args)


def _sc_agg_body(gA, gB, srcI, dstI, zrows, outA, outB,
                 src_v, dst_v, rows_v, acc, sem):
    c = lax.axis_index("c")
    s = lax.axis_index("s")

    pltpu.sync_copy(srcI.at[s], src_v)
    pltpu.sync_copy(dstI.at[s], dst_v)
    pltpu.sync_copy(zrows, acc.at[pl.ds(s * RPT, RPT)])
    plsc.subcore_barrier()

    def make_body(g_ref):
        def body(j, carry):
            pltpu.async_copy(g_ref.at[src_v.at[j]], rows_v, sem).wait()
            pltpu.sync_copy(rows_v, acc.at[dst_v.at[j]], add=True)
            return carry
        return body

    @pl.when(c == 0)
    def _():
        lax.fori_loop(0, EB, make_body(gA), 0)

    @pl.when(c == 1)
    def _():
        lax.fori_loop(0, EB, make_body(gB), 0)

    plsc.subcore_barrier()

    @pl.when(c == 0)
    def _():
        pltpu.sync_copy(acc.at[pl.ds(s * RPT, RPT)],
                        outA.at[pl.ds(s * RPT, RPT)])

    @pl.when(c == 1)
    def _():
        pltpu.sync_copy(acc.at[pl.ds(s * RPT, RPT)],
                        outB.at[pl.ds(s * RPT, RPT)])


@functools.cache
def _get_sc_agg1():
    return functools.partial(
        pl.kernel,
        mesh=_sc_mesh(),
        out_type=(
            jax.ShapeDtypeStruct((NPAD, W1), _f32),
            jax.ShapeDtypeStruct((NPAD, W1), _f32),
        ),
        scratch_types=[
            pltpu.VMEM((EB1, BLK), _i32),
            pltpu.VMEM((EB1, BLK), _i32),
            pltpu.VMEM((BLK, W1), _f32),
            pltpu.VMEM_SHARED((NPAD, W1), _f32),
            pltpu.SemaphoreType.DMA,
        ],
        compiler_params=pltpu.CompilerParams(use_tc_tiling_on_sc=False),
    )(_sc_agg1_body)


def _sc_agg1(*args):
    return _get_sc_agg1()(*args)


def _sc_agg1_body(xw, srcI, dstI, zrows, outA, outB,
                  src_v, dst_v, rows_v, acc, sem):
    c = lax.axis_index("c")
    s = lax.axis_index("s")
    wid = s * 2 + c

    pltpu.sync_copy(srcI.at[wid], src_v)
    pltpu.sync_copy(dstI.at[wid], dst_v)
    pltpu.sync_copy(zrows, acc.at[pl.ds(s * RPT, RPT)])
    plsc.subcore_barrier()

    def body(j, carry):
        pltpu.async_copy(xw.at[src_v.at[j]], rows_v, sem).wait()
        pltpu.sync_copy(rows_v, acc.at[dst_v.at[j]], add=True)
        return carry
    lax.fori_loop(0, EB1, body, 0)

    plsc.subcore_barrier()

    @pl.when(c == 0)
    def _():
        pltpu.sync_copy(acc.at[pl.ds(s * RPT, RPT)],
                        outA.at[pl.ds(s * RPT, RPT)])

    @pl.when(c == 1)
    def _():
        pltpu.sync_copy(acc.at[pl.ds(s * RPT, RPT)],
                        outB.at[pl.ds(s * RPT, RPT)])


_GRID = NPAD // RPT


def _rowspec(w=HH):
    return pl.BlockSpec((RPT, w), lambda i: (i, 0))


def _fullspec(shape):
    return pl.BlockSpec(shape, lambda i: tuple(0 for _ in shape))


def _row_mask(i):
    rows = i * RPT + lax.broadcasted_iota(_i32, (RPT, 1), 0)
    return rows < N


def _tc1_body(p0, p1, x_ref, Wr1, Wo1, b, Wr2, oA, oB, gA, gB):
    i = pl.program_id(0)
    agg = p0[...] + p1[...]
    y = jnp.dot(agg, Wr1[...], preferred_element_type=_f32)
    y = y + jnp.dot(x_ref[...], Wo1[...], preferred_element_type=_f32)
    y = jnp.maximum(y + b[...], 0.0)
    y = jnp.where(_row_mask(i), y, 0.0)
    oA[...] = y[:, :HH]
    oB[...] = y[:, HH:]
    g = jnp.dot(y, Wr2[...], preferred_element_type=_f32)
    gA[...] = g[:, :HH]
    gB[...] = g[:, HH:]


def _tc_layer1(p0, p1, xp, Wr1p, Wo1p, b1, Wr2):
    return pl.pallas_call(
        _tc1_body,
        grid=(_GRID,),
        in_specs=[_rowspec(W1), _rowspec(W1), _rowspec(W1),
                  _fullspec((W1, H)), _fullspec((W1, H)),
                  _fullspec((1, H)), _fullspec((H, H))],
        out_specs=[_rowspec()] * 4,
        out_shape=[jax.ShapeDtypeStruct((NPAD, HH), _f32)] * 4,
    )(p0, p1, xp, Wr1p, Wo1p, b1.reshape(1, H), Wr2)


def _tc_layer_body(aA, aB, hA, hB, WoA, WoB, b, Wrn, oA, oB, gA, gB):
    i = pl.program_id(0)
    y = jnp.concatenate([aA[...], aB[...]], axis=1)
    y = y + jnp.dot(hA[...], WoA[...], preferred_element_type=_f32)
    y = y + jnp.dot(hB[...], WoB[...], preferred_element_type=_f32)
    y = jnp.maximum(y + b[...], 0.0)
    y = jnp.where(_row_mask(i), y, 0.0)
    oA[...] = y[:, :HH]
    oB[...] = y[:, HH:]
    g = jnp.dot(y, Wrn[...], preferred_element_type=_f32)
    gA[...] = g[:, :HH]
    gB[...] = g[:, HH:]


def _tc_layer(aA, aB, hA, hB, Wo, b, Wrn):
    return pl.pallas_call(
        _tc_layer_body,
        grid=(_GRID,),
        in_specs=[_rowspec()] * 4 + [_fullspec((HH, H))] * 2
        + [_fullspec((1, H)), _fullspec((H, H))],
        out_specs=[_rowspec()] * 4,
        out_shape=[jax.ShapeDtypeStruct((NPAD, HH), _f32)] * 4,
    )(aA, aB, hA, hB, Wo[:HH], Wo[HH:], b.reshape(1, H), Wrn)


def _head_body(batch_ref, a4A, a4B, h3A, h3B, h1A, h1B, h2A, h2B,
               Wo4A, Wo4B, b4,
               Wf1, bf1, Wf2, bf2, Wf3, bf3, Wf4, bf4,
               out_ref, S, C):
    i = pl.program_id(0)

    @pl.when(i == 0)
    def _():
        S[...] = jnp.zeros_like(S)
        C[...] = jnp.zeros_like(C)

    y4 = jnp.concatenate([a4A[...], a4B[...]], axis=1)
    y4 = y4 + jnp.dot(h3A[...], Wo4A[...], preferred_element_type=_f32)
    y4 = y4 + jnp.dot(h3B[...], Wo4B[...], preferred_element_type=_f32)
    y4 = jnp.maximum(y4 + b4[...], 0.0)

    b_ids = batch_ref[0, 0, :]
    onehot = (lax.broadcasted_iota(_i32, (G, RPT), 0)
              == b_ids[None, :]).astype(_f32)
    parts = [h1A[...], h1B[...], h2A[...], h2B[...], h3A[...], h3B[...],
             y4[:, :HH], y4[:, HH:]]
    for k, hr in enumerate(parts):
        S[:, k * HH:(k + 1) * HH] += jnp.dot(
            onehot, hr, preferred_element_type=_f32)
    C[...] += jnp.broadcast_to(
        jnp.sum(onehot, axis=1, keepdims=True), (G, HH))

    @pl.when(i == _GRID - 1)
    def _():
        cnt = C[:, 0:1]
        pooled = S[...] / jnp.maximum(cnt, 1.0)
        t = jnp.maximum(
            jnp.dot(pooled, Wf1[...], preferred_element_type=_f32)
            + bf1[...], 0.0)
        t = jnp.maximum(
            jnp.dot(t, Wf2[...], preferred_element_type=_f32)
            + bf2[...], 0.0)
        t = jnp.maximum(
            jnp.dot(t, Wf3[...], preferred_element_type=_f32)
            + bf3[...], 0.0)
        out_ref[...] = (jnp.dot(t, Wf4[...], preferred_element_type=_f32)
                        + bf4[...])


def _head(batchp, a4A, a4B, h3A, h3B, h1A, h1B, h2A, h2B, Wo4, b4,
          Wf1, bf1, Wf2, bf2, Wf3, bf3, Wf4, bf4):
    args = [batchp, a4A, a4B, h3A, h3B, h1A, h1B, h2A, h2B,
            Wo4[:HH], Wo4[HH:], b4.reshape(1, H),
            Wf1, bf1.reshape(1, H), Wf2, bf2.reshape(1, H),
            Wf3, bf3.reshape(1, H), Wf4, bf4.reshape(1, 1)]
    in_specs = (
        [pl.BlockSpec((1, 1, RPT), lambda i: (i, 0, 0))]
        + [_rowspec()] * 8
        + [_fullspec((HH, H)), _fullspec((HH, H)), _fullspec((1, H)),
           _fullspec((4 * H, H)), _fullspec((1, H)),
           _fullspec((H, H)), _fullspec((1, H)),
           _fullspec((H, H)), _fullspec((1, H)),
           _fullspec((H, 1)), _fullspec((1, 1))])
    return pl.pallas_call(
        _head_body,
        grid=(_GRID,),
        in_specs=in_specs,
        out_specs=pl.BlockSpec((G, 1), lambda i: (0, 0)),
        out_shape=jax.ShapeDtypeStruct((G, 1), _f32),
        scratch_shapes=[pltpu.VMEM((G, 4 * H), _f32),
                        pltpu.VMEM((G, HH), _f32)],
    )(*args)


def kernel(x, edge_index, batch,
           Wr1, Wo1, b1, Wr2, Wo2, b2, Wr3, Wo3, b3, Wr4, Wo4, b4,
           Wf1, bf1, Wf2, bf2, Wf3, bf3, Wf4, bf4):
    xp = jnp.zeros((NPAD, W1), _f32).at[:N, :F0].set(x)
    Wr1p = jnp.zeros((W1, H), _f32).at[:F0].set(Wr1)
    Wo1p = jnp.zeros((W1, H), _f32).at[:F0].set(Wo1)

    src = edge_index[0]
    dst = edge_index[1]
    padi = jnp.full((NTILES, EB * BLK - EPT), N, _i32)
    srcp = jnp.concatenate([src.reshape(NTILES, EPT), padi],
                           axis=1).reshape(NTILES, EB, BLK)
    dstp = jnp.concatenate([dst.reshape(NTILES, EPT), padi],
                           axis=1).reshape(NTILES, EB, BLK)
    padi1 = jnp.full((32, EB1 * BLK - EPT1), N, _i32)
    srcp1 = jnp.concatenate([src.reshape(32, EPT1), padi1],
                            axis=1).reshape(32, EB1, BLK)
    dstp1 = jnp.concatenate([dst.reshape(32, EPT1), padi1],
                            axis=1).reshape(32, EB1, BLK)
    zrows = jnp.zeros((RPT, HH), _f32)
    zrows1 = jnp.zeros((RPT, W1), _f32)
    batchp = jnp.full((NPAD,), G, _i32).at[:N].set(batch) \
                .reshape(_GRID, 1, RPT)

    p0, p1 = _sc_agg1(xp, srcp1, dstp1, zrows1)
    h1A, h1B, g2A, g2B = _tc_layer1(p0, p1, xp, Wr1p, Wo1p, b1, Wr2)

    a2A, a2B = _sc_agg(g2A, g2B, srcp, dstp, zrows)
    h2A, h2B, g3A, g3B = _tc_layer(a2A, a2B, h1A, h1B, Wo2, b2, Wr3)

    a3A, a3B = _sc_agg(g3A, g3B, srcp, dstp, zrows)
    h3A, h3B, g4A, g4B = _tc_layer(a3A, a3B, h2A, h2B, Wo3, b3, Wr4)

    a4A, a4B = _sc_agg(g4A, g4B, srcp, dstp, zrows)
    out = _head(batchp, a4A, a4B, h3A, h3B, h1A, h1B, h2A, h2B, Wo4, b4,
                Wf1, bf1, Wf2, bf2, Wf3, bf3, Wf4, bf4)
    return out.reshape(-1)


# speedup vs baseline: 1.2487x; 1.2487x over previous
import functools

import jax
import jax.numpy as jnp
from jax import lax
from jax.experimental import pallas as pl
from jax.experimental.pallas import tpu as pltpu
from jax.experimental.pallas import tpu_sc as plsc

N = 10000
E = 160000
G = 64
F0 = 28
H = 256
HH = 128
W1 = 32
NPAD = 10112
RPT = NPAD // 16
NTILES = 16
BLK = 128
EB = 79
EPT = E // NTILES
EB1 = 40
EPT1 = E // 32

_f32 = jnp.float32
_i32 = jnp.int32


def _sc_mesh():
    return plsc.VectorSubcoreMesh(core_axis_name="c", subcore_axis_name="s",
                                  num_cores=2, num_subcores=16)


@functools.cache
def _get_sc_agg():
    return functools.partial(
        pl.kernel,
        mesh=_sc_mesh(),
        out_type=(
            jax.ShapeDtypeStruct((NPAD, HH), _f32),
            jax.ShapeDtypeStruct((NPAD, HH), _f32),
        ),
        scratch_types=[
            pltpu.VMEM((EB, BLK), _i32),
            pltpu.VMEM((EB, BLK), _i32),
            pltpu.VMEM((BLK, HH), _f32),
            pltpu.VMEM_SHARED((NPAD, HH), _f32),
            pltpu.SemaphoreType.DMA,
        ],
    )(_sc_agg_body)


def _sc_agg(*args):
    return _get_sc_agg()(*args)


def _sc_agg_body(gA, gB, srcI, dstI, zrows, outA, outB,
                 src_v, dst_v, rows_v, acc, sem):
    c = lax.axis_index("c")
    s = lax.axis_index("s")

    pltpu.sync_copy(srcI.at[s], src_v)
    pltpu.sync_copy(dstI.at[s], dst_v)
    pltpu.sync_copy(zrows, acc.at[pl.ds(s * RPT, RPT)])
    plsc.subcore_barrier()

    def make_body(g_ref):
        def body(j, carry):
            pltpu.async_copy(g_ref.at[src_v.at[j]], rows_v, sem).wait()
            pltpu.sync_copy(rows_v, acc.at[dst_v.at[j]], add=True)
            return carry
        return body

    @pl.when(c == 0)
    def _():
        lax.fori_loop(0, EB, make_body(gA), 0)

    @pl.when(c == 1)
    def _():
        lax.fori_loop(0, EB, make_body(gB), 0)

    plsc.subcore_barrier()

    @pl.when(c == 0)
    def _():
        pltpu.sync_copy(acc.at[pl.ds(s * RPT, RPT)],
                        outA.at[pl.ds(s * RPT, RPT)])

    @pl.when(c == 1)
    def _():
        pltpu.sync_copy(acc.at[pl.ds(s * RPT, RPT)],
                        outB.at[pl.ds(s * RPT, RPT)])


@functools.cache
def _get_sc_agg1():
    return functools.partial(
        pl.kernel,
        mesh=_sc_mesh(),
        out_type=(
            jax.ShapeDtypeStruct((NPAD, W1), _f32),
            jax.ShapeDtypeStruct((NPAD, W1), _f32),
        ),
        scratch_types=[
            pltpu.VMEM((EB1, BLK), _i32),
            pltpu.VMEM((EB1, BLK), _i32),
            pltpu.VMEM((BLK, W1), _f32),
            pltpu.VMEM_SHARED((NPAD, W1), _f32),
            pltpu.SemaphoreType.DMA,
        ],
        compiler_params=pltpu.CompilerParams(use_tc_tiling_on_sc=False),
    )(_sc_agg1_body)


def _sc_agg1(*args):
    return _get_sc_agg1()(*args)


def _sc_agg1_body(xw, srcI, dstI, zrows, outA, outB,
                  src_v, dst_v, rows_v, acc, sem):
    c = lax.axis_index("c")
    s = lax.axis_index("s")
    wid = s * 2 + c

    pltpu.sync_copy(srcI.at[wid], src_v)
    pltpu.sync_copy(dstI.at[wid], dst_v)
    pltpu.sync_copy(zrows, acc.at[pl.ds(s * RPT, RPT)])
    plsc.subcore_barrier()

    def body(j, carry):
        pltpu.async_copy(xw.at[src_v.at[j]], rows_v, sem).wait()
        pltpu.sync_copy(rows_v, acc.at[dst_v.at[j]], add=True)
        return carry
    lax.fori_loop(0, EB1, body, 0)

    plsc.subcore_barrier()

    @pl.when(c == 0)
    def _():
        pltpu.sync_copy(acc.at[pl.ds(s * RPT, RPT)],
                        outA.at[pl.ds(s * RPT, RPT)])

    @pl.when(c == 1)
    def _():
        pltpu.sync_copy(acc.at[pl.ds(s * RPT, RPT)],
                        outB.at[pl.ds(s * RPT, RPT)])


_GRID = NPAD // RPT


def _rowspec(w=HH):
    return pl.BlockSpec((RPT, w), lambda i: (i, 0))


def _fullspec(shape):
    return pl.BlockSpec(shape, lambda i: tuple(0 for _ in shape))


def _row_mask(i):
    rows = i * RPT + lax.broadcasted_iota(_i32, (RPT, 1), 0)
    return rows < N


def _tc1_body(p0, p1, x_ref, Wr1, Wo1, b, Wr2, oA, oB, gA, gB):
    i = pl.program_id(0)
    agg = p0[...] + p1[...]
    y = jnp.dot(agg, Wr1[...], preferred_element_type=_f32)
    y = y + jnp.dot(x_ref[...], Wo1[...], preferred_element_type=_f32)
    y = jnp.maximum(y + b[...], 0.0)
    y = jnp.where(_row_mask(i), y, 0.0)
    oA[...] = y[:, :HH]
    oB[...] = y[:, HH:]
    g = jnp.dot(y, Wr2[...], preferred_element_type=_f32)
    gA[...] = g[:, :HH]
    gB[...] = g[:, HH:]


def _tc_layer1(p0, p1, xp, Wr1p, Wo1p, b1, Wr2):
    return pl.pallas_call(
        _tc1_body,
        grid=(_GRID,),
        in_specs=[_rowspec(W1), _rowspec(W1), _rowspec(W1),
                  _fullspec((W1, H)), _fullspec((W1, H)),
                  _fullspec((1, H)), _fullspec((H, H))],
        out_specs=[_rowspec()] * 4,
        out_shape=[jax.ShapeDtypeStruct((NPAD, HH), _f32)] * 4,
    )(p0, p1, xp, Wr1p, Wo1p, b1.reshape(1, H), Wr2)


def _tc_layer_body(aA, aB, hA, hB, WoA, WoB, b, Wrn, oA, oB, gA, gB):
    i = pl.program_id(0)
    y = jnp.concatenate([aA[...], aB[...]], axis=1)
    y = y + jnp.dot(hA[...], WoA[...], preferred_element_type=_f32)
    y = y + jnp.dot(hB[...], WoB[...], preferred_element_type=_f32)
    y = jnp.maximum(y + b[...], 0.0)
    y = jnp.where(_row_mask(i), y, 0.0)
    oA[...] = y[:, :HH]
    oB[...] = y[:, HH:]
    g = jnp.dot(y, Wrn[...], preferred_element_type=_f32)
    gA[...] = g[:, :HH]
    gB[...] = g[:, HH:]


def _tc_layer(aA, aB, hA, hB, Wo, b, Wrn):
    return pl.pallas_call(
        _tc_layer_body,
        grid=(_GRID,),
        in_specs=[_rowspec()] * 4 + [_fullspec((HH, H))] * 2
        + [_fullspec((1, H)), _fullspec((H, H))],
        out_specs=[_rowspec()] * 4,
        out_shape=[jax.ShapeDtypeStruct((NPAD, HH), _f32)] * 4,
    )(aA, aB, hA, hB, Wo[:HH], Wo[HH:], b.reshape(1, H), Wrn)


def _head_body(batch_ref, a4A, a4B, h3A, h3B, h1A, h1B, h2A, h2B,
               Wo4A, Wo4B, b4,
               Wf1, bf1, Wf2, bf2, Wf3, bf3, Wf4, bf4,
               out_ref, S, C):
    i = pl.program_id(0)

    @pl.when(i == 0)
    def _():
        S[...] = jnp.zeros_like(S)
        C[...] = jnp.zeros_like(C)

    y4 = jnp.concatenate([a4A[...], a4B[...]], axis=1)
    y4 = y4 + jnp.dot(h3A[...], Wo4A[...], preferred_element_type=_f32)
    y4 = y4 + jnp.dot(h3B[...], Wo4B[...], preferred_element_type=_f32)
    y4 = jnp.maximum(y4 + b4[...], 0.0)

    b_ids = batch_ref[0, 0, :]
    onehot = (lax.broadcasted_iota(_i32, (G, RPT), 0)
              == b_ids[None, :]).astype(_f32)
    parts = [h1A[...], h1B[...], h2A[...], h2B[...], h3A[...], h3B[...],
             y4[:, :HH], y4[:, HH:]]
    for k, hr in enumerate(parts):
        S[:, k * HH:(k + 1) * HH] += jnp.dot(
            onehot, hr, preferred_element_type=_f32)
    C[...] += jnp.broadcast_to(
        jnp.sum(onehot, axis=1, keepdims=True), (G, HH))

    @pl.when(i == _GRID - 1)
    def _():
        cnt = C[:, 0:1]
        pooled = S[...] / jnp.maximum(cnt, 1.0)
        t = jnp.maximum(
            jnp.dot(pooled, Wf1[...], preferred_element_type=_f32)
            + bf1[...], 0.0)
        t = jnp.maximum(
            jnp.dot(t, Wf2[...], preferred_element_type=_f32)
            + bf2[...], 0.0)
        t = jnp.maximum(
            jnp.dot(t, Wf3[...], preferred_element_type=_f32)
            + bf3[...], 0.0)
        out_ref[...] = (jnp.dot(t, Wf4[...], preferred_element_type=_f32)
                        + bf4[...])


def _head(batchp, a4A, a4B, h3A, h3B, h1A, h1B, h2A, h2B, Wo4, b4,
          Wf1, bf1, Wf2, bf2, Wf3, bf3, Wf4, bf4):
    args = [batchp, a4A, a4B, h3A, h3B, h1A, h1B, h2A, h2B,
            Wo4[:HH], Wo4[HH:], b4.reshape(1, H),
            Wf1, bf1.reshape(1, H), Wf2, bf2.reshape(1, H),
            Wf3, bf3.reshape(1, H), Wf4, bf4.reshape(1, 1)]
    in_specs = (
        [pl.BlockSpec((1, 1, RPT), lambda i: (i, 0, 0))]
        + [_rowspec()] * 8
        + [_fullspec((HH, H)), _fullspec((HH, H)), _fullspec((1, H)),
           _fullspec((4 * H, H)), _fullspec((1, H)),
           _fullspec((H, H)), _fullspec((1, H)),
           _fullspec((H, H)), _fullspec((1, H)),
           _fullspec((H, 1)), _fullspec((1, 1))])
    return pl.pallas_call(
        _head_body,
        grid=(_GRID,),
        in_specs=in_specs,
        out_specs=pl.BlockSpec((G, 1), lambda i: (0, 0)),
        out_shape=jax.ShapeDtypeStruct((G, 1), _f32),
        scratch_shapes=[pltpu.VMEM((G, 4 * H), _f32),
                        pltpu.VMEM((G, HH), _f32)],
    )(*args)


def kernel(x, edge_index, batch,
           Wr1, Wo1, b1, Wr2, Wo2, b2, Wr3, Wo3, b3, Wr4, Wo4, b4,
           Wf1, bf1, Wf2, bf2, Wf3, bf3, Wf4, bf4):
    xp = jnp.zeros((NPAD, W1), _f32).at[:N, :F0].set(x)
    Wr1p = jnp.zeros((W1, H), _f32).at[:F0].set(Wr1)
    Wo1p = jnp.zeros((W1, H), _f32).at[:F0].set(Wo1)

    src = edge_index[0]
    dst = edge_index[1]
    npad_e = EB * BLK - EPT
    spread = (jnp.arange(NTILES)[:, None] * 7
              + jnp.arange(npad_e)[None, :]) % (NPAD - N)
    padi_src = jnp.full((NTILES, npad_e), N, _i32)
    padi_dst = (N + spread).astype(_i32)
    srcp = jnp.concatenate([src.reshape(NTILES, EPT), padi_src],
                           axis=1).reshape(NTILES, EB, BLK)
    dstp = jnp.concatenate([dst.reshape(NTILES, EPT), padi_dst],
                           axis=1).reshape(NTILES, EB, BLK)
    npad_e1 = EB1 * BLK - EPT1
    spread1 = (jnp.arange(32)[:, None] * 13
               + jnp.arange(npad_e1)[None, :]) % (NPAD - N)
    padi1_src = jnp.full((32, npad_e1), N, _i32)
    padi1_dst = (N + spread1).astype(_i32)
    srcp1 = jnp.concatenate([src.reshape(32, EPT1), padi1_src],
                            axis=1).reshape(32, EB1, BLK)
    dstp1 = jnp.concatenate([dst.reshape(32, EPT1), padi1_dst],
                            axis=1).reshape(32, EB1, BLK)
    zrows = jnp.zeros((RPT, HH), _f32)
    zrows1 = jnp.zeros((RPT, W1), _f32)
    batchp = jnp.full((NPAD,), G, _i32).at[:N].set(batch) \
                .reshape(_GRID, 1, RPT)

    p0, p1 = _sc_agg1(xp, srcp1, dstp1, zrows1)
    h1A, h1B, g2A, g2B = _tc_layer1(p0, p1, xp, Wr1p, Wo1p, b1, Wr2)

    a2A, a2B = _sc_agg(g2A, g2B, srcp, dstp, zrows)
    h2A, h2B, g3A, g3B = _tc_layer(a2A, a2B, h1A, h1B, Wo2, b2, Wr3)

    a3A, a3B = _sc_agg(g3A, g3B, srcp, dstp, zrows)
    h3A, h3B, g4A, g4B = _tc_layer(a3A, a3B, h2A, h2B, Wo3, b3, Wr4)

    a4A, a4B = _sc_agg(g4A, g4B, srcp, dstp, zrows)
    out = _head(batchp, a4A, a4B, h3A, h3B, h1A, h1B, h2A, h2B, Wo4, b4,
                Wf1, bf1, Wf2, bf2, Wf3, bf3, Wf4, bf4)
    return out.reshape(-1)
